# baseline (speedup 1.0000x reference)
import functools

import jax
import jax.numpy as jnp
from jax.experimental import pallas as pl
from jax.experimental.pallas import tpu as pltpu

_DN_RHS_T = (((1,), (1,)), ((), ()))


def _q_kernel(query_ref, wq_ref, bq_ref, q_ref):
    q_ref[:] = jnp.tanh(
        jax.lax.dot_general(query_ref[:], wq_ref[:], _DN_RHS_T,
                            preferred_element_type=jnp.float32)
        + bq_ref[:])


def _main_kernel(groups_ref, wn_ref, bn_ref, q_ref, pscale_ref, out_ref,
                 gcopy_ref, dots_s, mask_s, wnt_s, *, gpb, steps):
    i = pl.program_id(0)
    g = groups_ref[:]
    gcopy_ref[:] = g
    m = g.shape[1]
    rows = gpb * m
    g2 = g.reshape(rows, g.shape[2])
    d = wnt_s.shape[1]

    @pl.when(i == 0)
    def _init():
        wnt_s[:] = wn_ref[:].T

    z = jnp.dot(g2, wnt_s[:], preferred_element_type=jnp.float32) + bn_ref[:]
    xq = jnp.tanh(z) * q_ref[:]
    dd = xq.reshape(gpb, m, d).sum(axis=2)
    dots_s[pl.ds(i * gpb, gpb), :] = dd
    gh = g[:, :, :128]
    lane = jax.lax.broadcasted_iota(jnp.int32, gh.shape, 2)
    first = jnp.sum(jnp.abs(gh) * (lane == 0).astype(jnp.float32), axis=2)
    mask_s[pl.ds(i * gpb, gpb), :] = (first != 0.0).astype(jnp.float32)

    @pl.when(i == steps - 1)
    def _finish():
        _softmax_finale(dots_s, pscale_ref, mask_s, out_ref)


def _softmax_finale(dots_s, pscale_ref, mask_s, out_ref):
    d = dots_s[:]
    m1 = jnp.max(d, axis=1, keepdims=True)
    e = jnp.exp(d - m1)
    a = e / jnp.sum(e, axis=1, keepdims=True)
    logits = a * pscale_ref[:]
    gm = jnp.max(logits, axis=(0, 1), keepdims=True)
    e2 = jnp.exp(logits - gm)
    w = e2 / jnp.sum(e2, axis=(0, 1), keepdims=True)
    out_ref[:] = w * mask_s[:]


def kernel(query, groups, probabilities, Wq, bq, Wn, bn):
    n, m, e = groups.shape
    d = Wq.shape[0]

    q = pl.pallas_call(
        _q_kernel,
        out_shape=jax.ShapeDtypeStruct((1, d), jnp.float32),
    )(query, Wq, bq.reshape(1, d))

    gpb = 8
    steps = n // gpb
    pscale = jnp.broadcast_to(probabilities.reshape(n, 1) * 10.0, (n, m))

    w, gcopy = pl.pallas_call(
        functools.partial(_main_kernel, gpb=gpb, steps=steps),
        grid=(steps,),
        in_specs=[
            pl.BlockSpec((gpb, m, e), lambda i: (i, 0, 0)),
            pl.BlockSpec((d, e), lambda i: (0, 0)),
            pl.BlockSpec((1, d), lambda i: (0, 0)),
            pl.BlockSpec((1, d), lambda i: (0, 0)),
            pl.BlockSpec((n, m), lambda i: (0, 0)),
        ],
        out_specs=[
            pl.BlockSpec((n, m), lambda i: (0, 0)),
            pl.BlockSpec((gpb, m, e), lambda i: (i, 0, 0)),
        ],
        out_shape=[
            jax.ShapeDtypeStruct((n, m), jnp.float32),
            jax.ShapeDtypeStruct((n, m, e), jnp.float32),
        ],
        scratch_shapes=[
            pltpu.VMEM((n, m), jnp.float32),
            pltpu.VMEM((n, m), jnp.float32),
            pltpu.VMEM((e, d), jnp.float32),
        ],
    )(groups, Wn, bn.reshape(1, d), q, pscale)

    return (w.reshape(n, m, 1), gcopy)


# speedup vs baseline: 1.0683x; 1.0683x over previous
import functools

import jax
import jax.numpy as jnp
from jax.experimental import pallas as pl
from jax.experimental.pallas import tpu as pltpu

_DN_RHS_T = (((1,), (1,)), ((), ()))


def _q_kernel(query_ref, wq_ref, bq_ref, q_ref):
    q_ref[:] = jnp.tanh(
        jax.lax.dot_general(query_ref[:], wq_ref[:], _DN_RHS_T,
                            preferred_element_type=jnp.float32)
        + bq_ref[:])


def _main_kernel(groups_ref, wn_ref, bn_ref, q_ref, pscale_ref, out_ref,
                 gcopy_ref, dots_s, mask_s, wnt_s, *, gpb, steps):
    i = pl.program_id(0)

    @pl.when(i == 0)
    def _init():
        wnt_s[:] = wn_ref[:].T

    g = groups_ref[:]
    gcopy_ref[:] = g
    m = g.shape[1]
    rows = gpb * m
    g2 = g.reshape(rows, g.shape[2])
    d = wnt_s.shape[1]

    z = jnp.dot(g2, wnt_s[:], preferred_element_type=jnp.float32) + bn_ref[:]
    xq = jnp.tanh(z) * q_ref[:]
    dd = xq.reshape(gpb, m, d).sum(axis=2)
    dots_s[pl.ds(i * gpb, gpb), :] = dd
    gh = g[:, :, :128]
    lane = jax.lax.broadcasted_iota(jnp.int32, gh.shape, 2)
    first = jnp.sum(jnp.abs(gh) * (lane == 0).astype(jnp.float32), axis=2)
    mask_s[pl.ds(i * gpb, gpb), :] = (first != 0.0).astype(jnp.float32)

    @pl.when(i == steps - 1)
    def _finish():
        _softmax_finale(dots_s, pscale_ref, mask_s, out_ref)


def _softmax_finale(dots_s, pscale_ref, mask_s, out_ref):
    d = dots_s[:]
    m1 = jnp.max(d, axis=1, keepdims=True)
    e = jnp.exp(d - m1)
    a = e / jnp.sum(e, axis=1, keepdims=True)
    logits = a * pscale_ref[:]
    gm = jnp.max(logits, axis=(0, 1), keepdims=True)
    e2 = jnp.exp(logits - gm)
    w = e2 / jnp.sum(e2, axis=(0, 1), keepdims=True)
    out_ref[:] = w * mask_s[:]


def kernel(query, groups, probabilities, Wq, bq, Wn, bn):
    n, m, e = groups.shape
    d = Wq.shape[0]

    q = pl.pallas_call(
        _q_kernel,
        out_shape=jax.ShapeDtypeStruct((1, d), jnp.float32),
    )(query, Wq, bq.reshape(1, d))

    gpb = 8
    steps = n // gpb
    pscale = jnp.broadcast_to(probabilities.reshape(n, 1) * 10.0, (n, m))

    w, gcopy = pl.pallas_call(
        functools.partial(_main_kernel, gpb=gpb, steps=steps),
        grid=(steps,),
        in_specs=[
            pl.BlockSpec((gpb, m, e), lambda i: (i, 0, 0)),
            pl.BlockSpec((d, e), lambda i: (0, 0)),
            pl.BlockSpec((1, d), lambda i: (0, 0)),
            pl.BlockSpec((1, d), lambda i: (0, 0)),
            pl.BlockSpec((n, m), lambda i: (0, 0)),
        ],
        out_specs=[
            pl.BlockSpec((n, m), lambda i: (0, 0)),
            pl.BlockSpec((gpb, m, e), lambda i: (i, 0, 0)),
        ],
        out_shape=[
            jax.ShapeDtypeStruct((n, m), jnp.float32),
            jax.ShapeDtypeStruct((n, m, e), jnp.float32),
        ],
        scratch_shapes=[
            pltpu.VMEM((n, m), jnp.float32),
            pltpu.VMEM((n, m), jnp.float32),
            pltpu.VMEM((e, d), jnp.float32),
        ],
    )(groups, Wn, bn.reshape(1, d), q, pscale)

    return (w.reshape(n, m, 1), gcopy)
